# baseline (speedup 1.0000x reference)
import functools

import jax
import jax.numpy as jnp
from jax.experimental import pallas as pl
from jax.experimental.pallas import tpu as pltpu

_NUM_USERS = 50000
_LATENT = 64
_R = 5
_N_LAYERS = 2
_NEG_SLOPE = 0.01
_MLP_BLK = 2000


def _leaky(x):
    return jnp.where(x > 0, x, _NEG_SLOPE * x)


def _att_mlp_body(x_ref, w1_ref, c_ref, w2_ref, b2_ref, o_ref):
    x = x_ref[...]
    h = jnp.dot(x, w1_ref[0], preferred_element_type=jnp.float32) + c_ref[0]
    h = _leaky(h)
    h = jnp.dot(h, w2_ref[0], preferred_element_type=jnp.float32) + b2_ref[0]
    o_ref[0] = h


def _att_mlp(x, w1a, c, w2, b2):
    n = x.shape[0]
    grid = (n // _MLP_BLK, _R)
    return pl.pallas_call(
        _att_mlp_body,
        grid=grid,
        in_specs=[
            pl.BlockSpec((_MLP_BLK, _LATENT), lambda i, r: (i, 0)),
            pl.BlockSpec((1, _LATENT, _LATENT), lambda i, r: (r, 0, 0)),
            pl.BlockSpec((1, 1, _LATENT), lambda i, r: (r, 0, 0)),
            pl.BlockSpec((1, _LATENT, _LATENT), lambda i, r: (r, 0, 0)),
            pl.BlockSpec((1, 1, _LATENT), lambda i, r: (r, 0, 0)),
        ],
        out_specs=pl.BlockSpec((1, _MLP_BLK, _LATENT), lambda i, r: (r, i, 0)),
        out_shape=jax.ShapeDtypeStruct((_R, n, _LATENT), jnp.float32),
    )(x, w1a, c, w2, b2)


def _pred_body(z_ref, w1_ref, b1_ref, w2_ref, b2_ref, w3_ref, b3_ref, o_ref):
    h = jnp.dot(z_ref[...], w1_ref[...], preferred_element_type=jnp.float32) + b1_ref[...]
    h = _leaky(h)
    h = jnp.dot(h, w2_ref[...], preferred_element_type=jnp.float32) + b2_ref[...]
    h = _leaky(h)
    o_ref[...] = jnp.dot(h, w3_ref[...], preferred_element_type=jnp.float32) + b3_ref[...]


def _pred(z, w1p, b1p, w2p, b2p, w3p, b3p):
    m = z.shape[0]
    return pl.pallas_call(
        _pred_body,
        out_shape=jax.ShapeDtypeStruct((m, 128), jnp.float32),
    )(z, w1p, b1p, w2p, b2p, w3p, b3p)


def _spmm_mean(h, graph_rows, graph_cols, graph_vals, n):
    agg = jnp.zeros((n, _LATENT), jnp.float32)
    for r in range(_R):
        msg = graph_vals[r][:, None] * jnp.take(h[r], graph_cols[r], axis=0)
        agg = agg + jax.ops.segment_sum(msg, graph_rows[r], num_segments=n)
    return agg / _R


def kernel(users, pos_items, neg_items, user_emb, item_emb, rating_emb,
           att_W1, att_b1, att_W2, att_b2,
           pred_W1, pred_b1, pred_W2, pred_b2, pred_W3, pred_b3,
           graph_rows, graph_cols, graph_vals):
    n = user_emb.shape[0] + item_emb.shape[0]

    w1a = att_W1[:, :_LATENT, :]
    w1b = att_W1[:, _LATENT:, :]
    t = rating_emb[1:_R + 1]
    c = (jnp.einsum("rk,rkj->rj", t, w1b) + att_b1)[:, None, :]
    b2 = att_b2[:, None, :]

    all_embs = jnp.concatenate([user_emb, item_emb], axis=0)
    acc = all_embs
    cur = all_embs
    for _ in range(_N_LAYERS):
        h = _att_mlp(cur, w1a, c, att_W2, b2)
        cur = _spmm_mean(h, graph_rows, graph_cols, graph_vals, n)
        acc = acc + cur
    light = acc / (_N_LAYERS + 1)

    users_e = jnp.take(light, users, axis=0)
    pos_e = jnp.take(light, _NUM_USERS + pos_items, axis=0)
    neg_e = jnp.take(light, _NUM_USERS + neg_items, axis=0)

    w1p = jnp.pad(pred_W1, ((0, 0), (0, 64)))
    b1p = jnp.pad(pred_b1, (0, 64))[None, :]
    w2p = jnp.pad(pred_W2, ((0, 64), (0, 96)))
    b2p = jnp.pad(pred_b2, (0, 96))[None, :]
    w3p = jnp.pad(pred_W3, ((0, 96), (0, 127)))
    b3p = jnp.pad(pred_b3, (0, 127))[None, :]

    z = jnp.concatenate(
        [jnp.concatenate([users_e, users_e], axis=0),
         jnp.concatenate([pos_e, neg_e], axis=0)], axis=1)
    out = _pred(z, w1p, b1p, w2p, b2p, w3p, b3p)[:, 0]
    b = users.shape[0]
    return (out[:b], out[b:])
